# baseline (speedup 1.0000x reference)
import jax
import jax.numpy as jnp
from jax.experimental import pallas as pl
from jax.experimental.pallas import tpu as pltpu

_CHUNK = 3328


def _body(ids_ref, x_ref, emb_ref, o_ref, b0, b1, si0, si1, so0, so1):
    row = emb_ref[ids_ref[0], :]
    bufs = (b0, b1)
    sin = (si0, si1)
    sout = (so0, so1)
    rows = x_ref.shape[0]
    n = pl.cdiv(rows, _CHUNK)

    def in_cp(g):
        lo = g * _CHUNK
        sz = min(_CHUNK, rows - lo)
        return pltpu.make_async_copy(
            x_ref.at[pl.ds(lo, sz)], bufs[g % 2].at[pl.ds(0, sz)], sin[g % 2])

    def out_cp(g):
        lo = g * _CHUNK
        sz = min(_CHUNK, rows - lo)
        return pltpu.make_async_copy(
            bufs[g % 2].at[pl.ds(0, sz)], o_ref.at[pl.ds(lo, sz)], sout[g % 2])

    in_cp(0).start()
    for g in range(n):
        if g + 1 < n:
            if g >= 1:
                out_cp(g - 1).wait()
            in_cp(g + 1).start()
        in_cp(g).wait()
        sz = min(_CHUNK, rows - g * _CHUNK)
        bufs[g % 2][pl.ds(0, sz), :] = bufs[g % 2][pl.ds(0, sz), :] + row[None, :]
        out_cp(g).start()
    if n >= 2:
        out_cp(n - 2).wait()
    out_cp(n - 1).wait()


def kernel(lang, emb_weight, dataset_id):
    b, s, d = lang.shape
    rows = b * s
    x = lang.reshape(rows, d)
    ids = jnp.asarray(dataset_id, jnp.int32).reshape(1)

    out = pl.pallas_call(
        _body,
        in_specs=[
            pl.BlockSpec(memory_space=pltpu.MemorySpace.SMEM),
            pl.BlockSpec(memory_space=pltpu.MemorySpace.HBM),
            pl.BlockSpec(memory_space=pltpu.MemorySpace.VMEM),
        ],
        out_specs=pl.BlockSpec(memory_space=pltpu.MemorySpace.HBM),
        out_shape=jax.ShapeDtypeStruct((rows, d), lang.dtype),
        scratch_shapes=[
            pltpu.VMEM((_CHUNK, d), jnp.float32),
            pltpu.VMEM((_CHUNK, d), jnp.float32),
            pltpu.SemaphoreType.DMA,
            pltpu.SemaphoreType.DMA,
            pltpu.SemaphoreType.DMA,
            pltpu.SemaphoreType.DMA,
        ],
    )(ids, x, emb_weight)
    return out.reshape(b, s, d)


# speedup vs baseline: 1.0155x; 1.0155x over previous
import jax
import jax.numpy as jnp
from jax.experimental import pallas as pl
from jax.experimental.pallas import tpu as pltpu

_BLOCK = 2944


def _body(ids_ref, x_ref, emb_ref, o_ref):
    row = emb_ref[ids_ref[0], :]
    o_ref[...] = x_ref[...] + row[None, :]


def kernel(lang, emb_weight, dataset_id):
    b, s, d = lang.shape
    n_vocab = emb_weight.shape[0]
    rows = b * s
    x = lang.reshape(rows, d)
    ids = jnp.asarray(dataset_id, jnp.int32).reshape(1)

    grid_spec = pltpu.PrefetchScalarGridSpec(
        num_scalar_prefetch=1,
        grid=(pl.cdiv(rows, _BLOCK),),
        in_specs=[
            pl.BlockSpec((_BLOCK, d), lambda i, ids: (i, 0)),
            pl.BlockSpec((n_vocab, d), lambda i, ids: (0, 0)),
        ],
        out_specs=pl.BlockSpec((_BLOCK, d), lambda i, ids: (i, 0)),
    )
    out = pl.pallas_call(
        _body,
        grid_spec=grid_spec,
        out_shape=jax.ShapeDtypeStruct((rows, d), lang.dtype),
        compiler_params=pltpu.CompilerParams(
            dimension_semantics=("parallel",),
        ),
    )(ids, x, emb_weight)
    return out.reshape(b, s, d)


# speedup vs baseline: 1.0179x; 1.0023x over previous
import jax
import jax.numpy as jnp
from jax.experimental import pallas as pl
from jax.experimental.pallas import tpu as pltpu

_BLOCK = 3200


def _body(ids_ref, x_ref, emb_ref, o_ref):
    row = emb_ref[ids_ref[0], :]
    o_ref[...] = x_ref[...] + row[None, :]


def kernel(lang, emb_weight, dataset_id):
    b, s, d = lang.shape
    n_vocab = emb_weight.shape[0]
    rows = b * s
    x = lang.reshape(rows, d)
    ids = jnp.asarray(dataset_id, jnp.int32).reshape(1)

    grid_spec = pltpu.PrefetchScalarGridSpec(
        num_scalar_prefetch=1,
        grid=(pl.cdiv(rows, _BLOCK),),
        in_specs=[
            pl.BlockSpec((_BLOCK, d), lambda i, ids: (i, 0)),
            pl.BlockSpec((n_vocab, d), lambda i, ids: (0, 0)),
        ],
        out_specs=pl.BlockSpec((_BLOCK, d), lambda i, ids: (i, 0)),
    )
    out = pl.pallas_call(
        _body,
        grid_spec=grid_spec,
        out_shape=jax.ShapeDtypeStruct((rows, d), lang.dtype),
        compiler_params=pltpu.CompilerParams(
            dimension_semantics=("parallel",),
        ),
    )(ids, x, emb_weight)
    return out.reshape(b, s, d)


# speedup vs baseline: 1.0251x; 1.0071x over previous
import jax
import jax.numpy as jnp
from jax.experimental import pallas as pl
from jax.experimental.pallas import tpu as pltpu

_BLOCK = 3072


def _body(ids_ref, x_ref, emb_ref, o_ref):
    row = emb_ref[ids_ref[0], :]
    o_ref[...] = x_ref[...] + row[None, :]


def kernel(lang, emb_weight, dataset_id):
    b, s, d = lang.shape
    n_vocab = emb_weight.shape[0]
    rows = b * s
    x = lang.reshape(rows, d)
    ids = jnp.asarray(dataset_id, jnp.int32).reshape(1)

    grid_spec = pltpu.PrefetchScalarGridSpec(
        num_scalar_prefetch=1,
        grid=(pl.cdiv(rows, _BLOCK),),
        in_specs=[
            pl.BlockSpec((_BLOCK, d), lambda i, ids: (i, 0)),
            pl.BlockSpec((n_vocab, d), lambda i, ids: (0, 0)),
        ],
        out_specs=pl.BlockSpec((_BLOCK, d), lambda i, ids: (i, 0)),
    )
    out = pl.pallas_call(
        _body,
        grid_spec=grid_spec,
        out_shape=jax.ShapeDtypeStruct((rows, d), lang.dtype),
        compiler_params=pltpu.CompilerParams(
            dimension_semantics=("parallel",),
        ),
    )(ids, x, emb_weight)
    return out.reshape(b, s, d)
